# baseline (speedup 1.0000x reference)
import functools

import jax
import jax.numpy as jnp
from jax.experimental import pallas as pl

NUM_EXPERTS = 64
TOPK = 8
ROUTE_SCALE = 1.0
PADDED_EXPERTS = 128
BLOCK_ROWS = 256


def _gate_kernel(x_ref, w_ref, vals_ref, idx_ref, scores_ref):
    x = x_ref[...]
    w = w_ref[...]
    s = jax.lax.dot_general(
        x, w, (((1,), (1,)), ((), ())), preferred_element_type=jnp.float32
    )
    col = jax.lax.broadcasted_iota(jnp.int32, s.shape, 1)
    s = jnp.where(col < NUM_EXPERTS, s, -jnp.inf)
    m = jnp.max(s, axis=-1, keepdims=True)
    e = jnp.exp(s - m)
    p = e / jnp.sum(e, axis=-1, keepdims=True)
    scores_ref[...] = p[:, :NUM_EXPERTS]

    vals = p
    out_vals = []
    out_idx = []
    for _ in range(TOPK):
        vmax = jnp.max(vals, axis=-1, keepdims=True)
        hit = vals == vmax
        idx = jnp.min(jnp.where(hit, col, PADDED_EXPERTS), axis=-1, keepdims=True)
        out_vals.append(vmax)
        out_idx.append(idx)
        vals = jnp.where(col == idx, -jnp.inf, vals)
    vals_ref[...] = jnp.concatenate(out_vals, axis=-1) * ROUTE_SCALE
    idx_ref[...] = jnp.concatenate(out_idx, axis=-1)


@jax.jit
def kernel(x, weight):
    rows = x.shape[0]
    blk = min(BLOCK_ROWS, rows)
    grid = (rows // blk,)
    w_pad = jnp.zeros((PADDED_EXPERTS, weight.shape[1]), jnp.float32).at[
        : weight.shape[0]
    ].set(weight)
    vals, idx, scores = pl.pallas_call(
        _gate_kernel,
        grid=grid,
        in_specs=[
            pl.BlockSpec((blk, x.shape[1]), lambda i: (i, 0)),
            pl.BlockSpec((PADDED_EXPERTS, weight.shape[1]), lambda i: (0, 0)),
        ],
        out_specs=[
            pl.BlockSpec((blk, TOPK), lambda i: (i, 0)),
            pl.BlockSpec((blk, TOPK), lambda i: (i, 0)),
            pl.BlockSpec((blk, NUM_EXPERTS), lambda i: (i, 0)),
        ],
        out_shape=[
            jax.ShapeDtypeStruct((rows, TOPK), jnp.float32),
            jax.ShapeDtypeStruct((rows, TOPK), jnp.int32),
            jax.ShapeDtypeStruct((rows, NUM_EXPERTS), jnp.float32),
        ],
    )(x, w_pad)
    return vals, idx, scores


# speedup vs baseline: 1.4749x; 1.4749x over previous
import jax
import jax.numpy as jnp
from jax.experimental import pallas as pl

NUM_EXPERTS = 64
TOPK = 8
ROUTE_SCALE = 1.0
BLOCK_ROWS = 256


def _gate_kernel(x_ref, w_ref, vals_ref, idx_ref, scores_ref):
    x = x_ref[...]
    w = w_ref[...]
    s = jax.lax.dot_general(
        w, x, (((1,), (1,)), ((), ())), preferred_element_type=jnp.float32
    )
    m = jnp.max(s, axis=0, keepdims=True)
    e = jnp.exp(s - m)
    p = e / jnp.sum(e, axis=0, keepdims=True)
    scores_ref[...] = p.T

    expert = jax.lax.broadcasted_iota(jnp.int32, p.shape, 0)
    vals = p
    out_vals = []
    out_idx = []
    for _ in range(TOPK):
        vmax = jnp.max(vals, axis=0, keepdims=True)
        hit = vals == vmax
        idx = jnp.min(jnp.where(hit, expert, NUM_EXPERTS), axis=0, keepdims=True)
        out_vals.append(vmax)
        out_idx.append(idx)
        vals = jnp.where(expert == idx, -jnp.inf, vals)
    vals_ref[...] = jnp.concatenate(out_vals, axis=0).T * ROUTE_SCALE
    idx_ref[...] = jnp.concatenate(out_idx, axis=0).T


@jax.jit
def kernel(x, weight):
    rows = x.shape[0]
    blk = min(BLOCK_ROWS, rows)
    grid = (rows // blk,)
    vals, idx, scores = pl.pallas_call(
        _gate_kernel,
        grid=grid,
        in_specs=[
            pl.BlockSpec((blk, x.shape[1]), lambda i: (i, 0)),
            pl.BlockSpec(weight.shape, lambda i: (0, 0)),
        ],
        out_specs=[
            pl.BlockSpec((blk, TOPK), lambda i: (i, 0)),
            pl.BlockSpec((blk, TOPK), lambda i: (i, 0)),
            pl.BlockSpec((blk, NUM_EXPERTS), lambda i: (i, 0)),
        ],
        out_shape=[
            jax.ShapeDtypeStruct((rows, TOPK), jnp.float32),
            jax.ShapeDtypeStruct((rows, TOPK), jnp.int32),
            jax.ShapeDtypeStruct((rows, NUM_EXPERTS), jnp.float32),
        ],
    )(x, weight)
    return vals, idx, scores
